# Initial kernel scaffold; baseline (speedup 1.0000x reference)
import functools

import jax
import jax.numpy as jnp
from jax import lax
from jax.experimental import pallas as pl
from jax.experimental.pallas import tpu as pltpu
from jax.experimental.pallas import tpu_sc as plsc

_NUM_GRAPHS = 128
_D_IN = 256
_D_HID = 512
_D_OUT = 256
_N_NODES = 10000

_NC = 2
_NS = 16
_NW = _NC * _NS
_RPW = 320
_RLAST = _N_NODES - (_NW - 1) * _RPW
_SUB = 40
_NSUB = _RPW // _SUB
_NSUB_LAST = _RLAST // _SUB


def _make_seg_pool():
  mesh = plsc.VectorSubcoreMesh(core_axis_name="c", subcore_axis_name="s")

  @functools.partial(
      pl.kernel,
      out_type=jax.ShapeDtypeStruct((_NC, _NUM_GRAPHS, _D_IN), jnp.float32),
      mesh=mesh,
      scratch_types=[
          pltpu.VMEM((_RPW, _D_IN), jnp.float32),
          pltpu.VMEM((_NSUB, _SUB), jnp.int32),
          pltpu.VMEM((8, _D_IN), jnp.float32),
          pltpu.VMEM_SHARED((_NUM_GRAPHS, _D_IN), jnp.float32),
          pltpu.SemaphoreType.DMA,
          pltpu.SemaphoreType.DMA,
      ],
  )
  def seg_pool(x_hbm, idx_hbm, out_hbm, xv, iv, zv, acc, sem_x, sem_i):
    cid = lax.axis_index("c")
    sid = lax.axis_index("s")
    wid = cid * _NS + sid
    base = wid * _RPW

    def issue(nsub):
      n = nsub * _SUB
      pltpu.async_copy(x_hbm.at[pl.ds(base, n), :], xv.at[pl.ds(0, n), :],
                       sem_x)
      pltpu.async_copy(idx_hbm.at[pl.ds(wid * _NSUB, nsub), :],
                       iv.at[pl.ds(0, nsub), :], sem_i)

    def drain_scatter(nsub):
      n = nsub * _SUB
      pltpu.make_async_copy(x_hbm.at[pl.ds(base, n), :],
                            xv.at[pl.ds(0, n), :], sem_x).wait()
      pltpu.make_async_copy(idx_hbm.at[pl.ds(wid * _NSUB, nsub), :],
                            iv.at[pl.ds(0, nsub), :], sem_i).wait()
      for t in range(nsub):
        pltpu.sync_copy(xv.at[pl.ds(t * _SUB, _SUB), :], acc.at[iv.at[t]],
                        add=True)

    last = _NW - 1

    @pl.when(wid < last)
    def _():
      issue(_NSUB)

    @pl.when(wid == last)
    def _():
      issue(_NSUB_LAST)

    z = jnp.zeros((16,), jnp.float32)
    for r in range(8):
      for c in range(_D_IN // 16):
        zv[r, pl.ds(c * 16, 16)] = z
    pltpu.sync_copy(zv, acc.at[pl.ds(sid * 8, 8), :])
    plsc.subcore_barrier()

    @pl.when(wid < last)
    def _():
      drain_scatter(_NSUB)

    @pl.when(wid == last)
    def _():
      drain_scatter(_NSUB_LAST)

    plsc.subcore_barrier()

    @pl.when(sid == 0)
    def _():
      pltpu.sync_copy(acc, out_hbm.at[cid])

  return seg_pool


_seg_pool = _make_seg_pool()


def _mlp_body(p_ref, w1_ref, b1_ref, w2_ref, b2_ref, o_ref):
  pooled = p_ref[0] + p_ref[1]
  h = jnp.dot(pooled, w1_ref[...], preferred_element_type=jnp.float32)
  h = h + b1_ref[...]
  h = jnp.where(h >= 0.0, h, 0.01 * h)
  o_ref[...] = (
      jnp.dot(h, w2_ref[...], preferred_element_type=jnp.float32)
      + b2_ref[...]
  )


def _mlp(partials, W1, b1, W2, b2):
  return pl.pallas_call(
      _mlp_body,
      out_shape=jax.ShapeDtypeStruct((_NUM_GRAPHS, _D_OUT), jnp.float32),
  )(partials, W1, b1.reshape(1, _D_HID), W2, b2.reshape(1, _D_OUT))


def kernel(x, edge_index, batch, W1, b1, W2, b2):
  del edge_index
  idx2 = batch.astype(jnp.int32).reshape(_N_NODES // _SUB, _SUB)
  partials = _seg_pool(x, idx2)
  return _mlp(partials, W1, b1, W2, b2)


# speedup vs baseline: 2.4471x; 2.4471x over previous
import functools

import jax
import jax.numpy as jnp
from jax import lax
from jax.experimental import pallas as pl
from jax.experimental.pallas import tpu as pltpu
from jax.experimental.pallas import tpu_sc as plsc

_NUM_GRAPHS = 128
_D_IN = 256
_D_HID = 512
_D_OUT = 256
_N_NODES = 10000

_NC = 2
_NS = 16
_NW = _NC * _NS
_RPW = 320
_RLAST = _N_NODES - (_NW - 1) * _RPW
_SUB = 40
_NSUB = _RPW // _SUB
_NSUB_LAST = _RLAST // _SUB


def _make_seg_pool():
  mesh = plsc.VectorSubcoreMesh(core_axis_name="c", subcore_axis_name="s")

  @functools.partial(
      pl.kernel,
      out_type=jax.ShapeDtypeStruct((_NW, _NUM_GRAPHS, _D_IN), jnp.float32),
      mesh=mesh,
      scratch_types=[
          pltpu.VMEM((_RPW, _D_IN), jnp.float32),
          pltpu.VMEM((_RPW,), jnp.int32),
          pltpu.VMEM((_NUM_GRAPHS, _D_IN), jnp.float32),
          pltpu.SemaphoreType.DMA,
          pltpu.SemaphoreType.DMA,
      ],
  )
  def seg_pool(x_hbm, idx_hbm, out_hbm, xv, iv, acc, sem_x, sem_i):
    cid = lax.axis_index("c")
    sid = lax.axis_index("s")
    wid = cid * _NS + sid
    base = wid * _RPW

    def issue(n):
      pltpu.async_copy(x_hbm.at[pl.ds(base, n), :], xv.at[pl.ds(0, n), :],
                       sem_x)
      pltpu.async_copy(idx_hbm.at[pl.ds(base, n)], iv.at[pl.ds(0, n)],
                       sem_i)

    def drain_accum(n):
      pltpu.make_async_copy(x_hbm.at[pl.ds(base, n), :],
                            xv.at[pl.ds(0, n), :], sem_x).wait()
      pltpu.make_async_copy(idx_hbm.at[pl.ds(base, n)], iv.at[pl.ds(0, n)],
                            sem_i).wait()

      def group_body(t, carry):
        gvec = iv[pl.ds(t * 16, 16)]
        for j in range(16):
          g = gvec[j]
          r = t * 16 + j
          for c in range(_D_IN // 16):
            v = xv[r, pl.ds(c * 16, 16)]
            plsc.addupdate(acc.at[g, pl.ds(c * 16, 16)], v)
        return carry

      lax.fori_loop(0, n // 16, group_body, 0)

    last = _NW - 1

    @pl.when(wid < last)
    def _():
      issue(_RPW)

    @pl.when(wid == last)
    def _():
      issue(_RLAST)

    z = jnp.zeros((16,), jnp.float32)

    def zero_row(r, carry):
      for c in range(_D_IN // 16):
        acc[r, pl.ds(c * 16, 16)] = z
      return carry

    lax.fori_loop(0, _NUM_GRAPHS, zero_row, 0)

    @pl.when(wid < last)
    def _():
      drain_accum(_RPW)

    @pl.when(wid == last)
    def _():
      drain_accum(_RLAST)

    pltpu.sync_copy(acc, out_hbm.at[wid])

  return seg_pool


_seg_pool = _make_seg_pool()


def _mlp_body(p_ref, w1_ref, b1_ref, w2_ref, b2_ref, o_ref):
  pooled = jnp.sum(p_ref[...], axis=0)
  h = jnp.dot(pooled, w1_ref[...], preferred_element_type=jnp.float32)
  h = h + b1_ref[...]
  h = jnp.where(h >= 0.0, h, 0.01 * h)
  o_ref[...] = (
      jnp.dot(h, w2_ref[...], preferred_element_type=jnp.float32)
      + b2_ref[...]
  )


def _mlp(partials, W1, b1, W2, b2):
  return pl.pallas_call(
      _mlp_body,
      out_shape=jax.ShapeDtypeStruct((_NUM_GRAPHS, _D_OUT), jnp.float32),
  )(partials, W1, b1.reshape(1, _D_HID), W2, b2.reshape(1, _D_OUT))


def kernel(x, edge_index, batch, W1, b1, W2, b2):
  del edge_index
  partials = _seg_pool(x, batch.astype(jnp.int32))
  return _mlp(partials, W1, b1, W2, b2)


# speedup vs baseline: 3.0660x; 1.2529x over previous
import functools

import jax
import jax.numpy as jnp
from jax import lax
from jax.experimental import pallas as pl
from jax.experimental.pallas import tpu as pltpu
from jax.experimental.pallas import tpu_sc as plsc

_NUM_GRAPHS = 128
_D_IN = 256
_D_HID = 512
_D_OUT = 256
_N_NODES = 10000

_NC = 2
_NS = 16
_NW = _NC * _NS
_RPW = 320
_RLAST = _N_NODES - (_NW - 1) * _RPW
_CHUNK = 80
_NCHUNK = _RPW // _CHUNK
_GROUPS = _CHUNK // 16


def _make_seg_pool():
  mesh = plsc.VectorSubcoreMesh(core_axis_name="c", subcore_axis_name="s")

  @functools.partial(
      pl.kernel,
      out_type=jax.ShapeDtypeStruct((_NW, _NUM_GRAPHS, _D_IN), jnp.float32),
      mesh=mesh,
      scratch_types=[
          pltpu.VMEM((_RPW, _D_IN), jnp.float32),
          pltpu.VMEM((_RPW,), jnp.int32),
          pltpu.VMEM((_NUM_GRAPHS, _D_IN), jnp.float32),
          pltpu.SemaphoreType.DMA,
          pltpu.SemaphoreType.DMA,
          pltpu.SemaphoreType.DMA,
      ],
  )
  def seg_pool(x_hbm, idx_hbm, out_hbm, xv, iv, acc, sem_a, sem_b, sem_i):
    cid = lax.axis_index("c")
    sid = lax.axis_index("s")
    wid = cid * _NS + sid
    base = wid * _RPW

    def x_copy(q, sem):
      off = q * _CHUNK
      return pltpu.make_async_copy(
          x_hbm.at[pl.ds(base + off, _CHUNK), :],
          xv.at[pl.ds(off, _CHUNK), :], sem)

    def idx_copy(n):
      return pltpu.make_async_copy(idx_hbm.at[pl.ds(base, n)],
                                   iv.at[pl.ds(0, n)], sem_i)

    def group_body(t, carry):
      gvec = iv[pl.ds(t * 16, 16)]
      gs = [gvec[j] for j in range(16)]
      for j in range(16):
        r = t * 16 + j
        vs = [xv[r, pl.ds(c * 16, 16)] for c in range(_D_IN // 16)]
        for c in range(_D_IN // 16):
          plsc.addupdate(acc.at[gs[j], pl.ds(c * 16, 16)], vs[c])
      return carry

    def process(q):
      lax.fori_loop(q * _GROUPS, (q + 1) * _GROUPS, group_body, 0)

    last = _NW - 1

    @pl.when(wid < last)
    def _():
      x_copy(0, sem_a).start()
      x_copy(1, sem_b).start()
      idx_copy(_RPW).start()

    @pl.when(wid == last)
    def _():
      x_copy(0, sem_a).start()
      idx_copy(_RLAST).start()

    z = jnp.zeros((16,), jnp.float32)

    def zero_row(r, carry):
      for c in range(_D_IN // 16):
        acc[r, pl.ds(c * 16, 16)] = z
      return carry

    lax.fori_loop(0, _NUM_GRAPHS, zero_row, 0)

    @pl.when(wid < last)
    def _():
      idx_copy(_RPW).wait()
      x_copy(0, sem_a).wait()
      process(0)
      x_copy(2, sem_a).start()
      x_copy(1, sem_b).wait()
      process(1)
      x_copy(3, sem_b).start()
      x_copy(2, sem_a).wait()
      process(2)
      x_copy(3, sem_b).wait()
      process(3)

    @pl.when(wid == last)
    def _():
      idx_copy(_RLAST).wait()
      x_copy(0, sem_a).wait()
      process(0)

    pltpu.sync_copy(acc, out_hbm.at[wid])

  return seg_pool


_seg_pool = _make_seg_pool()


def _mlp_body(p_ref, w1_ref, b1_ref, w2_ref, b2_ref, o_ref):
  pooled = jnp.sum(p_ref[...], axis=0)
  h = jnp.dot(pooled, w1_ref[...], preferred_element_type=jnp.float32)
  h = h + b1_ref[...]
  h = jnp.where(h >= 0.0, h, 0.01 * h)
  o_ref[...] = (
      jnp.dot(h, w2_ref[...], preferred_element_type=jnp.float32)
      + b2_ref[...]
  )


def _mlp(partials, W1, b1, W2, b2):
  return pl.pallas_call(
      _mlp_body,
      out_shape=jax.ShapeDtypeStruct((_NUM_GRAPHS, _D_OUT), jnp.float32),
  )(partials, W1, b1.reshape(1, _D_HID), W2, b2.reshape(1, _D_OUT))


def kernel(x, edge_index, batch, W1, b1, W2, b2):
  del edge_index
  partials = _seg_pool(x, batch.astype(jnp.int32))
  return _mlp(partials, W1, b1, W2, b2)
